# Initial kernel scaffold; baseline (speedup 1.0000x reference)
import jax
import jax.numpy as jnp
from jax.experimental import pallas as pl


def kernel(time, weekday, minute_embed, hour_embed, weekday_embed):
    raise NotImplementedError("write your pallas kernel here")


# speedup vs baseline: 10.1707x; 10.1707x over previous
import functools

import jax
import jax.numpy as jnp
from jax import lax
from jax.experimental import pallas as pl
from jax.experimental.pallas import tpu as pltpu
from jax.experimental.pallas import tpu_sc as plsc

D = 64
N_HOUR = 24
N_MIN = 4
N_TIME = N_HOUR * N_MIN
N_WDAY = 7
N_ROWS = N_TIME * N_WDAY

NUM_CORES = 2
NUM_SUBCORES = 16
NW = NUM_CORES * NUM_SUBCORES

CHUNK = 512
GATHER = 128
LANES = 16


def _table_body(h_ref, m_ref, w_ref, o_ref):
    r = lax.broadcasted_iota(jnp.int32, (N_ROWS, 1), 0)
    t = r // N_WDAY
    wd = r % N_WDAY
    h = t // N_MIN
    mn = t % N_MIN
    oh_h = (h == lax.broadcasted_iota(jnp.int32, (N_ROWS, N_HOUR), 1)).astype(
        jnp.float32
    )
    oh_m = (mn == lax.broadcasted_iota(jnp.int32, (N_ROWS, N_MIN), 1)).astype(
        jnp.float32
    )
    oh_w = (wd == lax.broadcasted_iota(jnp.int32, (N_ROWS, N_WDAY), 1)).astype(
        jnp.float32
    )
    o_ref[...] = (
        jnp.dot(oh_h, h_ref[...], preferred_element_type=jnp.float32)
        + jnp.dot(oh_m, m_ref[...], preferred_element_type=jnp.float32)
        + jnp.dot(oh_w, w_ref[...], preferred_element_type=jnp.float32)
    )


def _build_table(minute_embed, hour_embed, weekday_embed, interpret=False):
    return pl.pallas_call(
        _table_body,
        out_shape=jax.ShapeDtypeStruct((N_ROWS, D), jnp.float32),
        interpret=interpret,
    )(hour_embed, minute_embed, weekday_embed)


def _sc_gather(time_flat, weekday_flat, table):
    n = time_flat.shape[0]
    n_per_w = n // NW
    n_outer = n_per_w // CHUNK
    mesh = plsc.VectorSubcoreMesh(core_axis_name="c", subcore_axis_name="s")

    @functools.partial(
        pl.kernel,
        mesh=mesh,
        compiler_params=pltpu.CompilerParams(use_tc_tiling_on_sc=False),
        out_type=jax.ShapeDtypeStruct((n, D), jnp.float32),
        scratch_types=[
            pltpu.VMEM((CHUNK,), jnp.int32),
            pltpu.VMEM((CHUNK,), jnp.int32),
            pltpu.VMEM((CHUNK,), jnp.int32),
            pltpu.VMEM((CHUNK, D), jnp.float32),
            pltpu.SemaphoreType.DMA,
        ],
    )
    def body(time_hbm, wday_hbm, table_hbm, out_hbm, t_v, w_v, c_v, rows_v, sem):
        wid = lax.axis_index("s") * NUM_CORES + lax.axis_index("c")
        base = wid * n_per_w

        def outer(i, carry):
            off = base + i * CHUNK
            pltpu.sync_copy(time_hbm.at[pl.ds(off, CHUNK)], t_v)
            pltpu.sync_copy(wday_hbm.at[pl.ds(off, CHUNK)], w_v)

            def compute(j, carry2):
                sl = pl.ds(j * LANES, LANES)
                t = t_v[sl]
                w = w_v[sl]
                h = jnp.clip(t >> 2, 0, N_HOUR - 1)
                mn = t & 3
                wd = jnp.clip(w, 0, N_WDAY - 1)
                c_v[sl] = h * (N_MIN * N_WDAY) + mn * N_WDAY + wd
                return carry2

            lax.fori_loop(0, CHUNK // LANES, compute, 0)

            copies = []
            for g in range(CHUNK // GATHER):
                gs = pl.ds(g * GATHER, GATHER)
                copies.append(
                    pltpu.async_copy(table_hbm.at[c_v.at[gs]], rows_v.at[gs], sem)
                )
            for cp in copies:
                cp.wait()
            pltpu.sync_copy(rows_v, out_hbm.at[pl.ds(off, CHUNK)])
            return carry

        lax.fori_loop(0, n_outer, outer, 0)

    return body(time_flat, weekday_flat, table)


def kernel(time, weekday, minute_embed, hour_embed, weekday_embed):
    s, b = time.shape
    table = _build_table(minute_embed, hour_embed, weekday_embed)
    tf = time.reshape(-1).astype(jnp.int32)
    wf = weekday.reshape(-1).astype(jnp.int32)
    out = _sc_gather(tf, wf, table)
    return out.reshape(s, b, D)
